# Initial kernel scaffold; baseline (speedup 1.0000x reference)
import jax
import jax.numpy as jnp
from jax.experimental import pallas as pl


def kernel(x, codebooks):
    raise NotImplementedError("write your pallas kernel here")


# speedup vs baseline: 14.9961x; 14.9961x over previous
import functools

import jax
import jax.numpy as jnp
from jax import lax
from jax.experimental import pallas as pl
from jax.experimental.pallas import tpu as pltpu
from jax.experimental.pallas import tpu_sc as plsc

NUM_CODEBOOKS = 8
CODEBOOK_SIZE = 1024
CODEBOOK_DIM = 128
N_FRAMES = 131072

L = 16
NW = 32
FRAMES_PER_W = N_FRAMES // NW
CHUNK_F = 64
CHUNKS = FRAMES_PER_W // CHUNK_F
ROWS_PER_CHUNK = CHUNK_F * NUM_CODEBOOKS
IDX_ROWS = ROWS_PER_CHUNK // 128


def _body(table_hbm, idx_hbm, out_hbm, raw_v, idx_v, rows_v, acc_v, sem):
    wid = lax.axis_index("s") * 2 + lax.axis_index("c")
    offs = (lax.iota(jnp.int32, L) % NUM_CODEBOOKS) * CODEBOOK_SIZE

    def chunk_body(ch, _):
        fbase = wid * FRAMES_PER_W + ch * CHUNK_F
        pltpu.sync_copy(
            idx_hbm.at[pl.ds(fbase * NUM_CODEBOOKS, ROWS_PER_CHUNK)], raw_v)
        for i in range(IDX_ROWS):
            for j in range(128 // L):
                idx_v[i, pl.ds(j * L, L)] = (
                    raw_v[pl.ds(i * 128 + j * L, L)] + offs)
        copies = [
            pltpu.async_copy(
                table_hbm.at[idx_v.at[k]],
                rows_v.at[pl.ds(k * 128, 128)],
                sem,
            )
            for k in range(IDX_ROWS)
        ]
        for c in copies:
            c.wait()

        def frame_body(f, _):
            r = f * NUM_CODEBOOKS
            for d in range(CODEBOOK_DIM // L):
                sl = pl.ds(d * L, L)
                acc = rows_v[r, sl]
                for l in range(1, NUM_CODEBOOKS):
                    acc = acc + rows_v[r + l, sl]
                acc_v[f, sl] = acc
            return 0

        lax.fori_loop(0, CHUNK_F, frame_body, 0)
        pltpu.sync_copy(acc_v, out_hbm.at[pl.ds(fbase, CHUNK_F)])
        return 0

    lax.fori_loop(0, CHUNKS, chunk_body, 0)


@jax.jit
def _dequant(table, idx2d):
    mesh = plsc.VectorSubcoreMesh(core_axis_name="c", subcore_axis_name="s")
    f = pl.kernel(
        _body,
        mesh=mesh,
        out_type=jax.ShapeDtypeStruct((N_FRAMES, CODEBOOK_DIM), jnp.float32),
        scratch_types=[
            pltpu.VMEM((ROWS_PER_CHUNK,), jnp.int32),
            pltpu.VMEM((IDX_ROWS, 128), jnp.int32),
            pltpu.VMEM((ROWS_PER_CHUNK, CODEBOOK_DIM), jnp.float32),
            pltpu.VMEM((CHUNK_F, CODEBOOK_DIM), jnp.float32),
            pltpu.SemaphoreType.DMA,
        ],
    )
    return f(table, idx2d)


def kernel(x, codebooks):
    table = codebooks.reshape(NUM_CODEBOOKS * CODEBOOK_SIZE, CODEBOOK_DIM)
    idx_flat = x.reshape(N_FRAMES * NUM_CODEBOOKS)
    return _dequant(table, idx_flat)


# speedup vs baseline: 20.2526x; 1.3505x over previous
import jax
import jax.numpy as jnp
from jax import lax
from jax.experimental import pallas as pl
from jax.experimental.pallas import tpu as pltpu
from jax.experimental.pallas import tpu_sc as plsc

NUM_CODEBOOKS = 8
CODEBOOK_SIZE = 1024
CODEBOOK_DIM = 128
N_FRAMES = 131072

L = 16
NW = 32
FRAMES_PER_W = N_FRAMES // NW
CHUNK_F = 32
NCH = FRAMES_PER_W // CHUNK_F
ROWS = CHUNK_F * NUM_CODEBOOKS
IDX_ROWS = ROWS // 128


def _body(table_hbm, idx_hbm, out_hbm,
          raw0, raw1, idx0, idx1, rows0, rows1, acc0, acc1,
          gsem0, gsem1, osem0, osem1):
    raw = (raw0, raw1)
    idx = (idx0, idx1)
    rows = (rows0, rows1)
    acc = (acc0, acc1)
    gsem = (gsem0, gsem1)
    osem = (osem0, osem1)

    wid = lax.axis_index("s") * 2 + lax.axis_index("c")
    f0 = wid * FRAMES_PER_W
    offs = (lax.iota(jnp.int32, L) % NUM_CODEBOOKS) * CODEBOOK_SIZE

    def prep(ch, b):
        fbase = f0 + ch * CHUNK_F
        pltpu.sync_copy(idx_hbm.at[pl.ds(fbase * NUM_CODEBOOKS, ROWS)], raw[b])
        for i in range(IDX_ROWS):
            for j in range(128 // L):
                idx[b][i, pl.ds(j * L, L)] = (
                    raw[b][pl.ds(i * 128 + j * L, L)] + offs)
        for k in range(IDX_ROWS):
            pltpu.async_copy(table_hbm.at[idx[b].at[k]],
                             rows[b].at[pl.ds(k * 128, 128)], gsem[b])

    def drain_gather(b):
        pltpu.make_async_copy(table_hbm.at[pl.ds(0, ROWS)], rows[b],
                              gsem[b]).wait()

    def drain_out(b):
        pltpu.make_async_copy(acc[b], out_hbm.at[pl.ds(0, CHUNK_F)],
                              osem[b]).wait()

    def compute(b):
        def frame_body(f, _):
            r = f * NUM_CODEBOOKS
            for d in range(CODEBOOK_DIM // L):
                sl = pl.ds(d * L, L)
                a = rows[b][r, sl]
                for l in range(1, NUM_CODEBOOKS):
                    a = a + rows[b][r + l, sl]
                acc[b][f, sl] = a
            return 0

        lax.fori_loop(0, CHUNK_F, frame_body, 0)

    prep(0, 0)
    prep(1, 1)

    def pair_body(p, _):
        g = p * 2
        for b in range(2):
            ch = g + b
            drain_gather(b)
            compute(b)

            @pl.when(ch >= 2)
            def _():
                drain_out(b)

            pltpu.async_copy(acc[b],
                             out_hbm.at[pl.ds(f0 + ch * CHUNK_F, CHUNK_F)],
                             osem[b])

            @pl.when(ch + 2 < NCH)
            def _():
                prep(ch + 2, b)

        return 0

    lax.fori_loop(0, NCH // 2, pair_body, 0)
    drain_out(0)
    drain_out(1)


@jax.jit
def _dequant(table, idx_flat):
    mesh = plsc.VectorSubcoreMesh(core_axis_name="c", subcore_axis_name="s")
    f = pl.kernel(
        _body,
        mesh=mesh,
        out_type=jax.ShapeDtypeStruct((N_FRAMES, CODEBOOK_DIM), jnp.float32),
        scratch_types=[
            pltpu.VMEM((ROWS,), jnp.int32),
            pltpu.VMEM((ROWS,), jnp.int32),
            pltpu.VMEM((IDX_ROWS, 128), jnp.int32),
            pltpu.VMEM((IDX_ROWS, 128), jnp.int32),
            pltpu.VMEM((ROWS, CODEBOOK_DIM), jnp.float32),
            pltpu.VMEM((ROWS, CODEBOOK_DIM), jnp.float32),
            pltpu.VMEM((CHUNK_F, CODEBOOK_DIM), jnp.float32),
            pltpu.VMEM((CHUNK_F, CODEBOOK_DIM), jnp.float32),
            pltpu.SemaphoreType.DMA,
            pltpu.SemaphoreType.DMA,
            pltpu.SemaphoreType.DMA,
            pltpu.SemaphoreType.DMA,
        ],
    )
    return f(table, idx_flat)


def kernel(x, codebooks):
    table = codebooks.reshape(NUM_CODEBOOKS * CODEBOOK_SIZE, CODEBOOK_DIM)
    idx_flat = x.reshape(N_FRAMES * NUM_CODEBOOKS)
    return _dequant(table, idx_flat)
